# Initial kernel scaffold; baseline (speedup 1.0000x reference)
import functools

import jax
import jax.numpy as jnp
from jax import lax
from jax.experimental import pallas as pl
from jax.experimental.pallas import tpu as pltpu
from jax.experimental.pallas import tpu_sc as plsc

N_NODE = 10000
N_ALL = 2 * N_NODE
E = 160000
D = 128
H = 8
NEG = 0.2

NS = 16
EPT = E // NS
C = 80
NCHUNK = EPT // C
NPT = N_NODE // NS
WB = 125
ROWBLK = 1000
GRID = N_ALL // ROWBLK


def _leaky_v(x):
    return jnp.where(x >= 0, x, NEG * x)


def _tc_pre_body(x_ref, wfc_ref, al_ref, ar_ref, ae_ref, eemb_ref, wfce_ref,
                 h_ref, el_ref, er_ref, m8_ref, ee8_ref, mx_ref):
    i = pl.program_id(0)
    h = jnp.dot(x_ref[...], wfc_ref[...], preferred_element_type=jnp.float32)
    h_ref[...] = h
    el = jnp.dot(h, al_ref[...], preferred_element_type=jnp.float32)
    er = jnp.dot(h, ar_ref[...], preferred_element_type=jnp.float32)
    el_ref[...] = el
    er_ref[...] = er
    bm_el = jnp.max(el, axis=0, keepdims=True)
    bm_er = jnp.max(er, axis=0, keepdims=True)

    @pl.when(i == 0)
    def _():
        mx_ref[...] = jnp.full((4, H), -jnp.inf, jnp.float32)

    @pl.when(i < GRID // 2)
    def _():
        mx_ref[0:1, :] = jnp.maximum(mx_ref[0:1, :], bm_el)
        mx_ref[2:3, :] = jnp.maximum(mx_ref[2:3, :], bm_er)

    @pl.when(i >= GRID // 2)
    def _():
        mx_ref[1:2, :] = jnp.maximum(mx_ref[1:2, :], bm_el)
        mx_ref[3:4, :] = jnp.maximum(mx_ref[3:4, :], bm_er)

    @pl.when(i == GRID - 1)
    def _():
        eep = jnp.dot(eemb_ref[...], wfce_ref[...],
                      preferred_element_type=jnp.float32)
        eea = jnp.dot(eep, ae_ref[...], preferred_element_type=jnp.float32)
        ee8_ref[...] = eea
        m0 = mx_ref[0:1, :] + mx_ref[3:4, :] + eea[0:1, :]
        m1 = mx_ref[1:2, :] + mx_ref[2:3, :] + eea[1:2, :]
        m = jnp.concatenate([m0, m1], axis=0)
        m8_ref[...] = _leaky_v(m)


def _tc_pre(x_all, w_fc, al, ar, ae, edge_emb, w_fc_edge):
    return pl.pallas_call(
        _tc_pre_body,
        grid=(GRID,),
        in_specs=[
            pl.BlockSpec((ROWBLK, D), lambda i: (i, 0)),
            pl.BlockSpec((D, D), lambda i: (0, 0)),
            pl.BlockSpec((D, H), lambda i: (0, 0)),
            pl.BlockSpec((D, H), lambda i: (0, 0)),
            pl.BlockSpec((D, H), lambda i: (0, 0)),
            pl.BlockSpec((2, D), lambda i: (0, 0)),
            pl.BlockSpec((D, D), lambda i: (0, 0)),
        ],
        out_specs=[
            pl.BlockSpec((ROWBLK, D), lambda i: (i, 0)),
            pl.BlockSpec((ROWBLK, H), lambda i: (i, 0)),
            pl.BlockSpec((ROWBLK, H), lambda i: (i, 0)),
            pl.BlockSpec((2, H), lambda i: (0, 0)),
            pl.BlockSpec((2, H), lambda i: (0, 0)),
        ],
        out_shape=[
            jax.ShapeDtypeStruct((N_ALL, D), jnp.float32),
            jax.ShapeDtypeStruct((N_ALL, H), jnp.float32),
            jax.ShapeDtypeStruct((N_ALL, H), jnp.float32),
            jax.ShapeDtypeStruct((2, H), jnp.float32),
            jax.ShapeDtypeStruct((2, H), jnp.float32),
        ],
        scratch_shapes=[pltpu.VMEM((4, H), jnp.float32)],
    )(x_all, w_fc, al, ar, ae, edge_emb, w_fc_edge)


def _sc_body(el_h, er_h, h_h, src0_h, dst0_h, src1_h, dst1_h, m8_h, ee8_h,
             out_h,
             sidx, didx, didxo, elb, erb, hb, exb, m8v, ee8v, zb, zb8,
             t_sh, den_sh, sem_el, sem_er, sem_h):
    cid = lax.axis_index("c")
    sid = lax.axis_index("s")

    pltpu.sync_copy(m8_h, m8v)
    pltpu.sync_copy(ee8_h, ee8v)
    lane = lax.iota(jnp.int32, 16)
    col8 = lane % 8
    row01 = lane // 8
    rsel = jnp.full((16,), cid, jnp.int32)
    mv = plsc.load_gather(m8v, [rsel, col8])
    eev = plsc.load_gather(ee8v, [rsel, col8])
    zero16 = jnp.zeros((16,), jnp.float32)

    def _zb_row(i, c):
        for k in range(8):
            zb[i, pl.ds(16 * k, 16)] = zero16
        return c
    lax.fori_loop(0, WB, _zb_row, 0)

    def _zb8_row(i, c):
        rows = 2 * i + row01
        plsc.store_scatter(zb8, [rows, col8], zero16, mask=rows < WB)
        return c
    lax.fori_loop(0, WB // 2 + 1, _zb8_row, 0)

    for b in range(NPT // WB):
        rbase = sid * NPT + b * WB
        pltpu.sync_copy(zb, t_sh.at[pl.ds(rbase, WB)])
        pltpu.sync_copy(zb8, den_sh.at[pl.ds(rbase, WB)])
    plsc.subcore_barrier()

    def _relation(src_e, dst_e, src_off, dst_off):
        src_off_v = jnp.full((16,), src_off, jnp.int32)
        dst_off_v = jnp.full((16,), dst_off, jnp.int32)

        def _chunk(j, c):
            cb = sid * EPT + j * C
            pltpu.sync_copy(src_e.at[pl.ds(cb, C)], sidx)
            pltpu.sync_copy(dst_e.at[pl.ds(cb, C)], didx)
            for k in range(C // 16):
                sidx[pl.ds(16 * k, 16)] = sidx[pl.ds(16 * k, 16)] + src_off_v
                didxo[pl.ds(16 * k, 16)] = didx[pl.ds(16 * k, 16)] + dst_off_v
            cp_el = pltpu.async_copy(el_h.at[sidx], elb, sem_el)
            cp_er = pltpu.async_copy(er_h.at[didxo], erb, sem_er)
            cp_h = pltpu.async_copy(h_h.at[sidx], hb, sem_h)
            cp_el.wait()
            cp_er.wait()

            def _ex(i, c2):
                rows = 2 * i + row01
                el2 = plsc.load_gather(elb, [rows, col8])
                er2 = plsc.load_gather(erb, [rows, col8])
                z = el2 + er2 + eev
                ex = jnp.exp(jnp.where(z >= 0, z, NEG * z) - mv)
                plsc.store_scatter(exb, [rows, col8], ex)
                return c2
            lax.fori_loop(0, C // 2, _ex, 0)
            pltpu.sync_copy(exb, den_sh.at[didx], add=True)
            cp_h.wait()

            def _scale(r, c2):
                for hh in range(H):
                    sv = jnp.full((16,), exb[r, hh], jnp.float32)
                    hb[r, pl.ds(16 * hh, 16)] = hb[r, pl.ds(16 * hh, 16)] * sv
                return c2
            lax.fori_loop(0, C, _scale, 0)
            pltpu.sync_copy(hb, t_sh.at[didx], add=True)
            return c
        lax.fori_loop(0, NCHUNK, _chunk, 0)

    @pl.when(cid == 0)
    def _():
        _relation(src0_h, dst0_h, 0, N_NODE)

    @pl.when(cid == 1)
    def _():
        _relation(src1_h, dst1_h, N_NODE, 0)

    plsc.subcore_barrier()

    out_off = jnp.where(cid == 0, N_NODE, 0)
    for b in range(NPT // WB):
        rbase = sid * NPT + b * WB
        pltpu.sync_copy(t_sh.at[pl.ds(rbase, WB)], zb)
        pltpu.sync_copy(den_sh.at[pl.ds(rbase, WB)], zb8)

        def _div(r, c):
            for hh in range(H):
                dv = jnp.maximum(zb8[r, hh], 1e-12)
                zb[r, pl.ds(16 * hh, 16)] = zb[r, pl.ds(16 * hh, 16)] / dv
            return c
        lax.fori_loop(0, WB, _div, 0)
        pltpu.sync_copy(zb, out_h.at[pl.ds(out_off + rbase, WB)])


def _sc_messages(el_all, er_all, h_all, src0, dst0, src1, dst1, m8, ee8):
    mesh = plsc.VectorSubcoreMesh(core_axis_name="c", subcore_axis_name="s")
    return pl.kernel(
        _sc_body,
        out_type=jax.ShapeDtypeStruct((N_ALL, D), jnp.float32),
        mesh=mesh,
        scratch_types=[
            pltpu.VMEM((C,), jnp.int32),
            pltpu.VMEM((C,), jnp.int32),
            pltpu.VMEM((C,), jnp.int32),
            pltpu.VMEM((C, H), jnp.float32),
            pltpu.VMEM((C, H), jnp.float32),
            pltpu.VMEM((C, D), jnp.float32),
            pltpu.VMEM((C, H), jnp.float32),
            pltpu.VMEM((2, H), jnp.float32),
            pltpu.VMEM((2, H), jnp.float32),
            pltpu.VMEM((WB, D), jnp.float32),
            pltpu.VMEM((WB, H), jnp.float32),
            pltpu.VMEM_SHARED((N_NODE, D), jnp.float32),
            pltpu.VMEM_SHARED((N_NODE, H), jnp.float32),
            pltpu.SemaphoreType.DMA,
            pltpu.SemaphoreType.DMA,
            pltpu.SemaphoreType.DMA,
        ],
    )(el_all, er_all, h_all, src0, dst0, src1, dst1, m8, ee8)


def _tc_post_body(tn_ref, h_ref, g_ref, b_ref, o_ref):
    y = _leaky_v(tn_ref[...] + h_ref[...])
    mu = jnp.mean(y, axis=1, keepdims=True)
    d = y - mu
    var = jnp.mean(d * d, axis=1, keepdims=True)
    o_ref[...] = g_ref[...] * d * lax.rsqrt(var + 1e-5) + b_ref[...]


def _tc_post(tn_all, h_all, gamma, beta):
    return pl.pallas_call(
        _tc_post_body,
        grid=(GRID,),
        in_specs=[
            pl.BlockSpec((ROWBLK, D), lambda i: (i, 0)),
            pl.BlockSpec((ROWBLK, D), lambda i: (i, 0)),
            pl.BlockSpec((1, D), lambda i: (0, 0)),
            pl.BlockSpec((1, D), lambda i: (0, 0)),
        ],
        out_specs=pl.BlockSpec((ROWBLK, D), lambda i: (i, 0)),
        out_shape=jax.ShapeDtypeStruct((N_ALL, D), jnp.float32),
    )(tn_all, h_all, gamma, beta)


def kernel(x_author, x_paper, edge_writes, edge_written_by, W_fc, W_fc_edge,
           edge_emb, attn_l, attn_r, attn_e, gamma, beta):
    x_all = jnp.concatenate([x_author, x_paper], axis=0)
    eye = jnp.eye(H, dtype=jnp.float32)
    al = (eye[:, None, :] * attn_l[0][:, :, None]).reshape(D, H)
    ar = (eye[:, None, :] * attn_r[0][:, :, None]).reshape(D, H)
    ae = (eye[:, None, :] * attn_e[0][:, :, None]).reshape(D, H)

    h_all, el_all, er_all, m8, ee8 = _tc_pre(
        x_all, W_fc, al, ar, ae, edge_emb, W_fc_edge)

    tn_all = _sc_messages(
        el_all, er_all, h_all,
        edge_writes[0], edge_writes[1],
        edge_written_by[0], edge_written_by[1],
        m8, ee8)

    out_all = _tc_post(tn_all, h_all,
                       gamma.reshape(1, D), beta.reshape(1, D))
    return out_all[:N_NODE], out_all[N_NODE:]


# speedup vs baseline: 43.3404x; 43.3404x over previous
import functools

import jax
import jax.numpy as jnp
from jax import lax
from jax.experimental import pallas as pl
from jax.experimental.pallas import tpu as pltpu
from jax.experimental.pallas import tpu_sc as plsc

N_NODE = 10000
N_ALL = 2 * N_NODE
E = 160000
D = 128
H = 8
NEG = 0.2

NS = 16
EPT = E // NS
C = 80
NCHUNK = EPT // C
WBLK = 80
NBLK = N_NODE // WBLK
JMAX = (NBLK + NS - 1) // NS
ROWBLK = 1000
GRID = N_ALL // ROWBLK


def _leaky_v(x):
    return jnp.where(x >= 0, x, NEG * x)


def _tc_pre_body(x_ref, wfc_ref, al_ref, ar_ref, ae_ref, eemb_ref, wfce_ref,
                 h_ref, el_ref, er_ref, m8_ref, ee8_ref, mx_ref):
    i = pl.program_id(0)
    h = jnp.dot(x_ref[...], wfc_ref[...], preferred_element_type=jnp.float32)
    h_ref[...] = h
    el = jnp.dot(h, al_ref[...], preferred_element_type=jnp.float32)
    er = jnp.dot(h, ar_ref[...], preferred_element_type=jnp.float32)
    el_ref[...] = el
    er_ref[...] = er
    bm_el = jnp.max(el, axis=0, keepdims=True)
    bm_er = jnp.max(er, axis=0, keepdims=True)

    @pl.when(i == 0)
    def _():
        mx_ref[...] = jnp.full((4, H), -jnp.inf, jnp.float32)

    @pl.when(i < GRID // 2)
    def _():
        mx_ref[0:1, :] = jnp.maximum(mx_ref[0:1, :], bm_el)
        mx_ref[2:3, :] = jnp.maximum(mx_ref[2:3, :], bm_er)

    @pl.when(i >= GRID // 2)
    def _():
        mx_ref[1:2, :] = jnp.maximum(mx_ref[1:2, :], bm_el)
        mx_ref[3:4, :] = jnp.maximum(mx_ref[3:4, :], bm_er)

    @pl.when(i == GRID - 1)
    def _():
        eep = jnp.dot(eemb_ref[...], wfce_ref[...],
                      preferred_element_type=jnp.float32)
        eea = jnp.dot(eep, ae_ref[...], preferred_element_type=jnp.float32)
        ee8_ref[...] = eea
        m0 = mx_ref[0:1, :] + mx_ref[3:4, :] + eea[0:1, :]
        m1 = mx_ref[1:2, :] + mx_ref[2:3, :] + eea[1:2, :]
        m = jnp.concatenate([m0, m1], axis=0)
        m8_ref[...] = _leaky_v(m)


def _tc_pre(x_all, w_fc, al, ar, ae, edge_emb, w_fc_edge):
    return pl.pallas_call(
        _tc_pre_body,
        grid=(GRID,),
        in_specs=[
            pl.BlockSpec((ROWBLK, D), lambda i: (i, 0)),
            pl.BlockSpec((D, D), lambda i: (0, 0)),
            pl.BlockSpec((D, H), lambda i: (0, 0)),
            pl.BlockSpec((D, H), lambda i: (0, 0)),
            pl.BlockSpec((D, H), lambda i: (0, 0)),
            pl.BlockSpec((2, D), lambda i: (0, 0)),
            pl.BlockSpec((D, D), lambda i: (0, 0)),
        ],
        out_specs=[
            pl.BlockSpec((ROWBLK, D), lambda i: (i, 0)),
            pl.BlockSpec((ROWBLK, H), lambda i: (i, 0)),
            pl.BlockSpec((ROWBLK, H), lambda i: (i, 0)),
            pl.BlockSpec((2, H), lambda i: (0, 0)),
            pl.BlockSpec((2, H), lambda i: (0, 0)),
        ],
        out_shape=[
            jax.ShapeDtypeStruct((N_ALL, D), jnp.float32),
            jax.ShapeDtypeStruct((N_ALL, H), jnp.float32),
            jax.ShapeDtypeStruct((N_ALL, H), jnp.float32),
            jax.ShapeDtypeStruct((2, H), jnp.float32),
            jax.ShapeDtypeStruct((2, H), jnp.float32),
        ],
        scratch_shapes=[pltpu.VMEM((4, H), jnp.float32)],
    )(x_all, w_fc, al, ar, ae, edge_emb, w_fc_edge)


def _sc_body(el_h, er_h, h_h, src0_h, dst0_h, src1_h, dst1_h, m8_h, ee8_h,
             out_h,
             sidx, didx, didxo, elb, erb, hb, exb, m8v, ee8v, zb8,
             t_sh, den_sh, sem_el, sem_er, sem_h):
    cid = lax.axis_index("c")
    sid = lax.axis_index("s")

    pltpu.sync_copy(m8_h, m8v)
    pltpu.sync_copy(ee8_h, ee8v)
    lane = lax.iota(jnp.int32, 16)
    col8 = lane % 8
    row01 = lane // 8
    rsel = jnp.full((16,), cid, jnp.int32)
    mv = plsc.load_gather(m8v, [rsel, col8])
    eev = plsc.load_gather(ee8v, [rsel, col8])
    zero16 = jnp.zeros((16,), jnp.float32)

    def _zb_row(i, c):
        for k in range(8):
            hb[i, pl.ds(16 * k, 16)] = zero16
        return c
    lax.fori_loop(0, WBLK, _zb_row, 0)

    def _zb8_row(i, c):
        rows = 2 * i + row01
        plsc.store_scatter(zb8, [rows, col8], zero16)
        return c
    lax.fori_loop(0, WBLK // 2, _zb8_row, 0)

    def _zinit(j, c):
        bb = j * NS + sid

        @pl.when(bb < NBLK)
        def _():
            rbase = bb * WBLK
            pltpu.sync_copy(hb, t_sh.at[pl.ds(rbase, WBLK)])
            pltpu.sync_copy(zb8, den_sh.at[pl.ds(rbase, WBLK)])
        return c
    lax.fori_loop(0, JMAX, _zinit, 0)
    plsc.subcore_barrier()

    def _relation(src_e, dst_e, src_off, dst_off):
        src_off_v = jnp.full((16,), src_off, jnp.int32)
        dst_off_v = jnp.full((16,), dst_off, jnp.int32)

        def _chunk(j, c):
            cb = sid * EPT + j * C
            pltpu.sync_copy(src_e.at[pl.ds(cb, C)], sidx)
            pltpu.sync_copy(dst_e.at[pl.ds(cb, C)], didx)
            for k in range(C // 16):
                sidx[pl.ds(16 * k, 16)] = sidx[pl.ds(16 * k, 16)] + src_off_v
                didxo[pl.ds(16 * k, 16)] = didx[pl.ds(16 * k, 16)] + dst_off_v
            cp_el = pltpu.async_copy(el_h.at[sidx], elb, sem_el)
            cp_er = pltpu.async_copy(er_h.at[didxo], erb, sem_er)
            cp_h = pltpu.async_copy(h_h.at[sidx], hb, sem_h)
            cp_el.wait()
            cp_er.wait()

            def _ex(i, c2):
                rows = 2 * i + row01
                el2 = plsc.load_gather(elb, [rows, col8])
                er2 = plsc.load_gather(erb, [rows, col8])
                z = el2 + er2 + eev
                ex = jnp.exp(jnp.where(z >= 0, z, NEG * z) - mv)
                plsc.store_scatter(exb, [rows, col8], ex)
                return c2
            lax.fori_loop(0, C // 2, _ex, 0)
            pltpu.sync_copy(exb, den_sh.at[didx], add=True)
            cp_h.wait()

            def _scale(r, c2):
                rfull = jnp.full((16,), r, jnp.int32)
                for hh in range(H):
                    hful = jnp.full((16,), hh, jnp.int32)
                    sv = plsc.load_gather(exb, [rfull, hful])
                    hb[r, pl.ds(16 * hh, 16)] = hb[r, pl.ds(16 * hh, 16)] * sv
                return c2
            lax.fori_loop(0, C, _scale, 0)
            pltpu.sync_copy(hb, t_sh.at[didx], add=True)
            return c
        lax.fori_loop(0, NCHUNK, _chunk, 0)

    @pl.when(cid == 0)
    def _():
        _relation(src0_h, dst0_h, 0, N_NODE)

    @pl.when(cid == 1)
    def _():
        _relation(src1_h, dst1_h, N_NODE, 0)

    plsc.subcore_barrier()

    out_off = (1 - cid) * N_NODE

    def _wb(j, c):
        bb = j * NS + sid

        @pl.when(bb < NBLK)
        def _():
            rbase = bb * WBLK
            pltpu.sync_copy(t_sh.at[pl.ds(rbase, WBLK)], hb)
            pltpu.sync_copy(den_sh.at[pl.ds(rbase, WBLK)], zb8)

            def _div(r, c2):
                rfull = jnp.full((16,), r, jnp.int32)
                for hh in range(H):
                    hful = jnp.full((16,), hh, jnp.int32)
                    dv = jnp.maximum(
                        plsc.load_gather(zb8, [rfull, hful]), 1e-12)
                    hb[r, pl.ds(16 * hh, 16)] = hb[r, pl.ds(16 * hh, 16)] / dv
                return c2
            lax.fori_loop(0, WBLK, _div, 0)
            pltpu.sync_copy(hb, out_h.at[pl.ds(out_off + rbase, WBLK)])
        return c
    lax.fori_loop(0, JMAX, _wb, 0)


def _sc_messages(el_all, er_all, h_all, src0, dst0, src1, dst1, m8, ee8):
    mesh = plsc.VectorSubcoreMesh(core_axis_name="c", subcore_axis_name="s")
    return pl.kernel(
        _sc_body,
        out_type=jax.ShapeDtypeStruct((N_ALL, D), jnp.float32),
        mesh=mesh,
        compiler_params=pltpu.CompilerParams(
            needs_layout_passes=False, use_tc_tiling_on_sc=False),
        scratch_types=[
            pltpu.VMEM((C,), jnp.int32),
            pltpu.VMEM((C,), jnp.int32),
            pltpu.VMEM((C,), jnp.int32),
            pltpu.VMEM((C, H), jnp.float32),
            pltpu.VMEM((C, H), jnp.float32),
            pltpu.VMEM((C, D), jnp.float32),
            pltpu.VMEM((C, H), jnp.float32),
            pltpu.VMEM((2, H), jnp.float32),
            pltpu.VMEM((2, H), jnp.float32),
            pltpu.VMEM((WBLK, H), jnp.float32),
            pltpu.VMEM_SHARED((N_NODE, D), jnp.float32),
            pltpu.VMEM_SHARED((N_NODE, H), jnp.float32),
            pltpu.SemaphoreType.DMA,
            pltpu.SemaphoreType.DMA,
            pltpu.SemaphoreType.DMA,
        ],
    )(el_all, er_all, h_all, src0, dst0, src1, dst1, m8, ee8)


def _tc_post_body(tn_ref, h_ref, g_ref, b_ref, o_ref):
    y = _leaky_v(tn_ref[...] + h_ref[...])
    mu = jnp.mean(y, axis=1, keepdims=True)
    d = y - mu
    var = jnp.mean(d * d, axis=1, keepdims=True)
    o_ref[...] = g_ref[...] * d * lax.rsqrt(var + 1e-5) + b_ref[...]


def _tc_post(tn_all, h_all, gamma, beta):
    return pl.pallas_call(
        _tc_post_body,
        grid=(GRID,),
        in_specs=[
            pl.BlockSpec((ROWBLK, D), lambda i: (i, 0)),
            pl.BlockSpec((ROWBLK, D), lambda i: (i, 0)),
            pl.BlockSpec((1, D), lambda i: (0, 0)),
            pl.BlockSpec((1, D), lambda i: (0, 0)),
        ],
        out_specs=pl.BlockSpec((ROWBLK, D), lambda i: (i, 0)),
        out_shape=jax.ShapeDtypeStruct((N_ALL, D), jnp.float32),
    )(tn_all, h_all, gamma, beta)


def kernel(x_author, x_paper, edge_writes, edge_written_by, W_fc, W_fc_edge,
           edge_emb, attn_l, attn_r, attn_e, gamma, beta):
    x_all = jnp.concatenate([x_author, x_paper], axis=0)
    eye = jnp.eye(H, dtype=jnp.float32)
    al = (eye[:, None, :] * attn_l[0][:, :, None]).reshape(D, H)
    ar = (eye[:, None, :] * attn_r[0][:, :, None]).reshape(D, H)
    ae = (eye[:, None, :] * attn_e[0][:, :, None]).reshape(D, H)

    h_all, el_all, er_all, m8, ee8 = _tc_pre(
        x_all, W_fc, al, ar, ae, edge_emb, W_fc_edge)

    tn_all = _sc_messages(
        el_all, er_all, h_all,
        edge_writes[0], edge_writes[1],
        edge_written_by[0], edge_written_by[1],
        m8, ee8)

    out_all = _tc_post(tn_all, h_all,
                       gamma.reshape(1, D), beta.reshape(1, D))
    return out_all[:N_NODE], out_all[N_NODE:]


# speedup vs baseline: 56.1636x; 1.2959x over previous
import functools

import jax
import jax.numpy as jnp
from jax import lax
from jax.experimental import pallas as pl
from jax.experimental.pallas import tpu as pltpu
from jax.experimental.pallas import tpu_sc as plsc

N_NODE = 10000
N_ALL = 2 * N_NODE
E = 160000
D = 128
H = 8
NEG = 0.2

NS = 16
EPT = E // NS
C = 80
NCHUNK = EPT // C
WBLK = 80
NBLK = N_NODE // WBLK
JMAX = (NBLK + NS - 1) // NS
ROWBLK = 1000
GRID = N_ALL // ROWBLK


def _leaky_v(x):
    return jnp.where(x >= 0, x, NEG * x)


def _tc_pre_body(x_ref, wfc_ref, al_ref, ar_ref, ae_ref, eemb_ref, wfce_ref,
                 h_ref, el_ref, er_ref, m8_ref, ee8_ref, mx_ref):
    i = pl.program_id(0)
    h = jnp.dot(x_ref[...], wfc_ref[...], preferred_element_type=jnp.float32)
    h_ref[...] = h
    el = jnp.dot(h, al_ref[...], preferred_element_type=jnp.float32)
    er = jnp.dot(h, ar_ref[...], preferred_element_type=jnp.float32)
    el_ref[...] = el
    er_ref[...] = er
    bm_el = jnp.max(el, axis=0, keepdims=True)
    bm_er = jnp.max(er, axis=0, keepdims=True)

    @pl.when(i == 0)
    def _():
        mx_ref[...] = jnp.full((4, H), -jnp.inf, jnp.float32)

    @pl.when(i < GRID // 2)
    def _():
        mx_ref[0:1, :] = jnp.maximum(mx_ref[0:1, :], bm_el)
        mx_ref[2:3, :] = jnp.maximum(mx_ref[2:3, :], bm_er)

    @pl.when(i >= GRID // 2)
    def _():
        mx_ref[1:2, :] = jnp.maximum(mx_ref[1:2, :], bm_el)
        mx_ref[3:4, :] = jnp.maximum(mx_ref[3:4, :], bm_er)

    @pl.when(i == GRID - 1)
    def _():
        eep = jnp.dot(eemb_ref[...], wfce_ref[...],
                      preferred_element_type=jnp.float32)
        eea = jnp.dot(eep, ae_ref[...], preferred_element_type=jnp.float32)
        ee8_ref[...] = eea
        m0 = mx_ref[0:1, :] + mx_ref[3:4, :] + eea[0:1, :]
        m1 = mx_ref[1:2, :] + mx_ref[2:3, :] + eea[1:2, :]
        m = jnp.concatenate([m0, m1], axis=0)
        m8_ref[...] = _leaky_v(m)


def _tc_pre(x_all, w_fc, al, ar, ae, edge_emb, w_fc_edge):
    return pl.pallas_call(
        _tc_pre_body,
        grid=(GRID,),
        in_specs=[
            pl.BlockSpec((ROWBLK, D), lambda i: (i, 0)),
            pl.BlockSpec((D, D), lambda i: (0, 0)),
            pl.BlockSpec((D, H), lambda i: (0, 0)),
            pl.BlockSpec((D, H), lambda i: (0, 0)),
            pl.BlockSpec((D, H), lambda i: (0, 0)),
            pl.BlockSpec((2, D), lambda i: (0, 0)),
            pl.BlockSpec((D, D), lambda i: (0, 0)),
        ],
        out_specs=[
            pl.BlockSpec((ROWBLK, D), lambda i: (i, 0)),
            pl.BlockSpec((ROWBLK, H), lambda i: (i, 0)),
            pl.BlockSpec((ROWBLK, H), lambda i: (i, 0)),
            pl.BlockSpec((2, H), lambda i: (0, 0)),
            pl.BlockSpec((2, H), lambda i: (0, 0)),
        ],
        out_shape=[
            jax.ShapeDtypeStruct((N_ALL, D), jnp.float32),
            jax.ShapeDtypeStruct((N_ALL, H), jnp.float32),
            jax.ShapeDtypeStruct((N_ALL, H), jnp.float32),
            jax.ShapeDtypeStruct((2, H), jnp.float32),
            jax.ShapeDtypeStruct((2, H), jnp.float32),
        ],
        scratch_shapes=[pltpu.VMEM((4, H), jnp.float32)],
    )(x_all, w_fc, al, ar, ae, edge_emb, w_fc_edge)


def _sc_body(el_h, er_h, h_h, e0_h, e1_h, m8_h, ee8_h,
             out_h,
             idxb0, sidxo0, didxo0, didxs0, elb0, erb0, hb0, exb0,
             idxb1, sidxo1, didxo1, didxs1, elb1, erb1, hb1, exb1,
             m8v, ee8v, zb8, t_sh, den_sh,
             sem_ee0, sem_h0, sem_s0, sem_i0,
             sem_ee1, sem_h1, sem_s1, sem_i1):
    cid = lax.axis_index("c")
    sid = lax.axis_index("s")

    pltpu.sync_copy(m8_h, m8v)
    pltpu.sync_copy(ee8_h, ee8v)
    lane = lax.iota(jnp.int32, 16)
    col8 = lane % 8
    row01 = lane // 8
    rsel = jnp.full((16,), cid, jnp.int32)
    mv = plsc.load_gather(m8v, [rsel, col8])
    eev = plsc.load_gather(ee8v, [rsel, col8])
    zero16 = jnp.zeros((16,), jnp.float32)

    def _zb_row(i, c):
        for k in range(8):
            hb0[i, pl.ds(16 * k, 16)] = zero16
        return c
    lax.fori_loop(0, WBLK, _zb_row, 0)

    def _zb8_row(i, c):
        rows = 2 * i + row01
        plsc.store_scatter(zb8, [rows, col8], zero16)
        return c
    lax.fori_loop(0, WBLK // 2, _zb8_row, 0)

    def _zinit(j, c):
        bb = j * NS + sid

        @pl.when(bb < NBLK)
        def _():
            rbase = bb * WBLK
            pltpu.sync_copy(hb0, t_sh.at[pl.ds(rbase, WBLK)])
            pltpu.sync_copy(zb8, den_sh.at[pl.ds(rbase, WBLK)])
        return c
    lax.fori_loop(0, JMAX, _zinit, 0)
    plsc.subcore_barrier()

    slots = ((idxb0, sidxo0, didxo0, didxs0, elb0, erb0, hb0, exb0,
              sem_ee0, sem_h0, sem_s0, sem_i0),
             (idxb1, sidxo1, didxo1, didxs1, elb1, erb1, hb1, exb1,
              sem_ee1, sem_h1, sem_s1, sem_i1))

    def _relation(src_e, src_off, dst_off):
        src_off_v = jnp.full((16,), src_off, jnp.int32)
        dst_off_v = jnp.full((16,), dst_off, jnp.int32)

        def _issue_idx(j, s):
            idxb = slots[s][0]
            cb = sid * EPT + j * C
            return pltpu.async_copy(
                src_e.at[:, pl.ds(cb, C)], idxb, slots[s][11])

        def _wait_idx(s):
            idxb = slots[s][0]
            pltpu.make_async_copy(
                src_e.at[:, pl.ds(0, C)], idxb, slots[s][11]).wait()

        def _prep(s):
            idxb, sidxo, didxo, didxs = slots[s][0:4]
            for k in range(C // 16):
                dsl = pl.ds(16 * k, 16)
                sidxo[dsl] = idxb[0, dsl] + src_off_v
                didxo[dsl] = idxb[1, dsl] + dst_off_v
                didxs[dsl] = idxb[1, dsl]

        def _issue_gathers(s):
            _, sidxo, didxo, _, elb, erb, hb, _, sem_ee, sem_h = slots[s][:10]
            pltpu.async_copy(el_h.at[sidxo], elb, sem_ee)
            pltpu.async_copy(er_h.at[didxo], erb, sem_ee)
            pltpu.async_copy(h_h.at[sidxo], hb, sem_h)

        def _wait_gathers_ee(s):
            _, sidxo, didxo, _, elb, erb, _, _, sem_ee = slots[s][:9]
            pltpu.make_async_copy(el_h.at[sidxo], elb, sem_ee).wait()
            pltpu.make_async_copy(er_h.at[didxo], erb, sem_ee).wait()

        def _wait_gather_h(s):
            _, sidxo, _, _, _, _, hb, _, _, sem_h = slots[s][:10]
            pltpu.make_async_copy(h_h.at[sidxo], hb, sem_h).wait()

        def _drain_scatters(s):
            _, _, _, didxs, _, _, hb, exb, _, _, sem_s = slots[s][:11]
            pltpu.make_async_copy(exb, den_sh.at[didxs], sem_s).wait()
            pltpu.make_async_copy(hb, t_sh.at[didxs], sem_s).wait()

        def _compute(s):
            _, _, _, didxs, elb, erb, hb, exb, _, _, sem_s = slots[s][:11]
            _wait_gathers_ee(s)

            def _ex(i, c2):
                rows = 2 * i + row01
                el2 = plsc.load_gather(elb, [rows, col8])
                er2 = plsc.load_gather(erb, [rows, col8])
                z = el2 + er2 + eev
                ex = jnp.exp(jnp.where(z >= 0, z, NEG * z) - mv)
                plsc.store_scatter(exb, [rows, col8], ex)
                return c2
            lax.fori_loop(0, C // 2, _ex, 0)
            pltpu.async_copy(exb, den_sh.at[didxs], sem_s, add=True)
            _wait_gather_h(s)

            def _scale(r, c2):
                rfull = jnp.full((16,), r, jnp.int32)
                for hh in range(H):
                    hful = jnp.full((16,), hh, jnp.int32)
                    sv = plsc.load_gather(exb, [rfull, hful])
                    hb[r, pl.ds(16 * hh, 16)] = hb[r, pl.ds(16 * hh, 16)] * sv
                return c2
            lax.fori_loop(0, C, _scale, 0)
            pltpu.async_copy(hb, t_sh.at[didxs], sem_s, add=True)

        _issue_idx(0, 0).wait()
        _prep(0)
        _issue_gathers(0)
        _issue_idx(1, 1)

        npair = (NCHUNK + 1) // 2

        def _pair(jj, c):
            a = 2 * jj
            b = a + 1

            @pl.when(jj >= 1)
            def _():
                _drain_scatters(1)

            @pl.when(b < NCHUNK)
            def _():
                _wait_idx(1)
                _prep(1)
                _issue_gathers(1)
            _compute(0)

            @pl.when(jj < npair - 1)
            def _():
                _issue_idx(a + 2, 0)

            @pl.when(b < NCHUNK)
            def _():
                _compute(1)

                @pl.when(b + 2 < NCHUNK)
                def _():
                    _issue_idx(b + 2, 1)

            @pl.when(jj < npair - 1)
            def _():
                _drain_scatters(0)
                _wait_idx(0)
                _prep(0)
                _issue_gathers(0)
            return c
        lax.fori_loop(0, npair, _pair, 0)
        _drain_scatters(0)

    @pl.when(cid == 0)
    def _():
        _relation(e0_h, 0, N_NODE)

    @pl.when(cid == 1)
    def _():
        _relation(e1_h, N_NODE, 0)

    plsc.subcore_barrier()

    out_off = (1 - cid) * N_NODE

    def _wb(j, c):
        bb = j * NS + sid

        @pl.when(bb < NBLK)
        def _():
            rbase = bb * WBLK
            pltpu.sync_copy(t_sh.at[pl.ds(rbase, WBLK)], hb0)
            pltpu.sync_copy(den_sh.at[pl.ds(rbase, WBLK)], zb8)

            def _div(r, c2):
                rfull = jnp.full((16,), r, jnp.int32)
                for hh in range(H):
                    hful = jnp.full((16,), hh, jnp.int32)
                    dv = jnp.maximum(
                        plsc.load_gather(zb8, [rfull, hful]), 1e-12)
                    hb0[r, pl.ds(16 * hh, 16)] = (
                        hb0[r, pl.ds(16 * hh, 16)] / dv)
                return c2
            lax.fori_loop(0, WBLK, _div, 0)
            pltpu.sync_copy(hb0, out_h.at[pl.ds(out_off + rbase, WBLK)])
        return c
    lax.fori_loop(0, JMAX, _wb, 0)


def _sc_messages(el_all, er_all, h_all, e0, e1, m8, ee8):
    mesh = plsc.VectorSubcoreMesh(core_axis_name="c", subcore_axis_name="s")
    slot = [
        pltpu.VMEM((2, C), jnp.int32),
        pltpu.VMEM((C,), jnp.int32),
        pltpu.VMEM((C,), jnp.int32),
        pltpu.VMEM((C,), jnp.int32),
        pltpu.VMEM((C, H), jnp.float32),
        pltpu.VMEM((C, H), jnp.float32),
        pltpu.VMEM((C, D), jnp.float32),
        pltpu.VMEM((C, H), jnp.float32),
    ]
    return pl.kernel(
        _sc_body,
        out_type=jax.ShapeDtypeStruct((N_ALL, D), jnp.float32),
        mesh=mesh,
        compiler_params=pltpu.CompilerParams(
            needs_layout_passes=False, use_tc_tiling_on_sc=False),
        scratch_types=[
            *slot, *slot,
            pltpu.VMEM((2, H), jnp.float32),
            pltpu.VMEM((2, H), jnp.float32),
            pltpu.VMEM((WBLK, H), jnp.float32),
            pltpu.VMEM_SHARED((N_NODE, D), jnp.float32),
            pltpu.VMEM_SHARED((N_NODE, H), jnp.float32),
            pltpu.SemaphoreType.DMA, pltpu.SemaphoreType.DMA,
            pltpu.SemaphoreType.DMA, pltpu.SemaphoreType.DMA,
            pltpu.SemaphoreType.DMA, pltpu.SemaphoreType.DMA,
            pltpu.SemaphoreType.DMA, pltpu.SemaphoreType.DMA,
        ],
    )(el_all, er_all, h_all, e0, e1, m8, ee8)


def _tc_post_body(tn_ref, h_ref, g_ref, b_ref, o_ref):
    y = _leaky_v(tn_ref[...] + h_ref[...])
    mu = jnp.mean(y, axis=1, keepdims=True)
    d = y - mu
    var = jnp.mean(d * d, axis=1, keepdims=True)
    o_ref[...] = g_ref[...] * d * lax.rsqrt(var + 1e-5) + b_ref[...]


def _tc_post(tn_all, h_all, gamma, beta):
    return pl.pallas_call(
        _tc_post_body,
        grid=(GRID,),
        in_specs=[
            pl.BlockSpec((ROWBLK, D), lambda i: (i, 0)),
            pl.BlockSpec((ROWBLK, D), lambda i: (i, 0)),
            pl.BlockSpec((1, D), lambda i: (0, 0)),
            pl.BlockSpec((1, D), lambda i: (0, 0)),
        ],
        out_specs=pl.BlockSpec((ROWBLK, D), lambda i: (i, 0)),
        out_shape=jax.ShapeDtypeStruct((N_ALL, D), jnp.float32),
    )(tn_all, h_all, gamma, beta)


def kernel(x_author, x_paper, edge_writes, edge_written_by, W_fc, W_fc_edge,
           edge_emb, attn_l, attn_r, attn_e, gamma, beta):
    x_all = jnp.concatenate([x_author, x_paper], axis=0)
    eye = jnp.eye(H, dtype=jnp.float32)
    al = (eye[:, None, :] * attn_l[0][:, :, None]).reshape(D, H)
    ar = (eye[:, None, :] * attn_r[0][:, :, None]).reshape(D, H)
    ae = (eye[:, None, :] * attn_e[0][:, :, None]).reshape(D, H)

    h_all, el_all, er_all, m8, ee8 = _tc_pre(
        x_all, W_fc, al, ar, ae, edge_emb, W_fc_edge)

    tn_all = _sc_messages(
        el_all, er_all, h_all, edge_writes, edge_written_by, m8, ee8)

    out_all = _tc_post(tn_all, h_all,
                       gamma.reshape(1, D), beta.reshape(1, D))
    return out_all[:N_NODE], out_all[N_NODE:]


# speedup vs baseline: 156.3481x; 2.7838x over previous
import functools

import jax
import jax.numpy as jnp
from jax import lax
from jax.experimental import pallas as pl
from jax.experimental.pallas import tpu as pltpu
from jax.experimental.pallas import tpu_sc as plsc

N_NODE = 10000
N_ALL = 2 * N_NODE
E = 160000
D = 128
H = 8
NEG = 0.2

NS = 16
EPT = E // NS
C = 80
NCHUNK = EPT // C
WBLK = 80
NBLK = N_NODE // WBLK
JMAX = (NBLK + NS - 1) // NS
ROWBLK = 1000
GRID = N_ALL // ROWBLK


def _leaky_v(x):
    return jnp.where(x >= 0, x, NEG * x)


def _tc_pre_body(x_ref, wfc_ref, al_ref, ar_ref, ae_ref, eemb_ref, wfce_ref,
                 h_ref, el_ref, er_ref, m8_ref, ee8_ref, mx_ref):
    i = pl.program_id(0)
    h = jnp.dot(x_ref[...], wfc_ref[...], preferred_element_type=jnp.float32)
    h_ref[...] = h
    el = jnp.dot(h, al_ref[...], preferred_element_type=jnp.float32)
    er = jnp.dot(h, ar_ref[...], preferred_element_type=jnp.float32)
    el_ref[...] = el
    er_ref[...] = er
    bm_el = jnp.max(el, axis=0, keepdims=True)
    bm_er = jnp.max(er, axis=0, keepdims=True)

    @pl.when(i == 0)
    def _():
        mx_ref[...] = jnp.full((4, H), -jnp.inf, jnp.float32)

    @pl.when(i < GRID // 2)
    def _():
        mx_ref[0:1, :] = jnp.maximum(mx_ref[0:1, :], bm_el)
        mx_ref[2:3, :] = jnp.maximum(mx_ref[2:3, :], bm_er)

    @pl.when(i >= GRID // 2)
    def _():
        mx_ref[1:2, :] = jnp.maximum(mx_ref[1:2, :], bm_el)
        mx_ref[3:4, :] = jnp.maximum(mx_ref[3:4, :], bm_er)

    @pl.when(i == GRID - 1)
    def _():
        eep = jnp.dot(eemb_ref[...], wfce_ref[...],
                      preferred_element_type=jnp.float32)
        eea = jnp.dot(eep, ae_ref[...], preferred_element_type=jnp.float32)
        ee8_ref[...] = eea
        m0 = mx_ref[0:1, :] + mx_ref[3:4, :] + eea[0:1, :]
        m1 = mx_ref[1:2, :] + mx_ref[2:3, :] + eea[1:2, :]
        m = jnp.concatenate([m0, m1], axis=0)
        m8_ref[...] = _leaky_v(m)


def _tc_pre(x_all, w_fc, al, ar, ae, edge_emb, w_fc_edge):
    return pl.pallas_call(
        _tc_pre_body,
        grid=(GRID,),
        in_specs=[
            pl.BlockSpec((ROWBLK, D), lambda i: (i, 0)),
            pl.BlockSpec((D, D), lambda i: (0, 0)),
            pl.BlockSpec((D, H), lambda i: (0, 0)),
            pl.BlockSpec((D, H), lambda i: (0, 0)),
            pl.BlockSpec((D, H), lambda i: (0, 0)),
            pl.BlockSpec((2, D), lambda i: (0, 0)),
            pl.BlockSpec((D, D), lambda i: (0, 0)),
        ],
        out_specs=[
            pl.BlockSpec((ROWBLK, D), lambda i: (i, 0)),
            pl.BlockSpec((ROWBLK, H), lambda i: (i, 0)),
            pl.BlockSpec((ROWBLK, H), lambda i: (i, 0)),
            pl.BlockSpec((2, H), lambda i: (0, 0)),
            pl.BlockSpec((2, H), lambda i: (0, 0)),
        ],
        out_shape=[
            jax.ShapeDtypeStruct((N_ALL, D), jnp.float32),
            jax.ShapeDtypeStruct((N_ALL, H), jnp.float32),
            jax.ShapeDtypeStruct((N_ALL, H), jnp.float32),
            jax.ShapeDtypeStruct((2, H), jnp.float32),
            jax.ShapeDtypeStruct((2, H), jnp.float32),
        ],
        scratch_shapes=[pltpu.VMEM((4, H), jnp.float32)],
    )(x_all, w_fc, al, ar, ae, edge_emb, w_fc_edge)


def _sc_body(el_h, er_h, h_h, e0_h, e1_h, m8_h, ee8_h,
             out_h,
             idxb0, sidxo0, didxo0, didxs0, elb0, erb0, hb0, exb0,
             idxb1, sidxo1, didxo1, didxs1, elb1, erb1, hb1, exb1,
             m8v, ee8v, zb8, t_sh, den_sh,
             sem_ee0, sem_h0, sem_s0, sem_i0,
             sem_ee1, sem_h1, sem_s1, sem_i1):
    cid = lax.axis_index("c")
    sid = lax.axis_index("s")

    pltpu.sync_copy(m8_h, m8v)
    pltpu.sync_copy(ee8_h, ee8v)
    lane = lax.iota(jnp.int32, 16)
    col8 = lane % 8
    row01 = lane // 8
    rsel = jnp.full((16,), cid, jnp.int32)
    mv = plsc.load_gather(m8v, [rsel, col8])
    eev = plsc.load_gather(ee8v, [rsel, col8])
    zero16 = jnp.zeros((16,), jnp.float32)

    def _zb_row(i, c):
        for k in range(8):
            hb0[i, pl.ds(16 * k, 16)] = zero16
        return c
    lax.fori_loop(0, WBLK, _zb_row, 0)

    def _zb8_row(i, c):
        rows = 2 * i + row01
        plsc.store_scatter(zb8, [rows, col8], zero16)
        return c
    lax.fori_loop(0, WBLK // 2, _zb8_row, 0)

    def _zinit(j, c):
        bb = j * NS + sid

        @pl.when(bb < NBLK)
        def _():
            rbase = bb * WBLK
            pltpu.sync_copy(hb0, t_sh.at[pl.ds(rbase, WBLK)])
            pltpu.sync_copy(zb8, den_sh.at[pl.ds(rbase, WBLK)])
        return c
    lax.fori_loop(0, JMAX, _zinit, 0)
    plsc.subcore_barrier()

    slots = ((idxb0, sidxo0, didxo0, didxs0, elb0, erb0, hb0, exb0,
              sem_ee0, sem_h0, sem_s0, sem_i0),
             (idxb1, sidxo1, didxo1, didxs1, elb1, erb1, hb1, exb1,
              sem_ee1, sem_h1, sem_s1, sem_i1))

    def _relation(src_e, src_off, dst_off):
        src_off_v = jnp.full((16,), src_off, jnp.int32)
        dst_off_v = jnp.full((16,), dst_off, jnp.int32)

        def _issue_idx(j, s):
            idxb = slots[s][0]
            cb = sid * EPT + j * C
            return pltpu.async_copy(
                src_e.at[:, pl.ds(cb, C)], idxb, slots[s][11])

        def _wait_idx(s):
            idxb = slots[s][0]
            pltpu.make_async_copy(
                src_e.at[:, pl.ds(0, C)], idxb, slots[s][11]).wait()

        def _prep(s):
            idxb, sidxo, didxo, didxs = slots[s][0:4]
            for k in range(C // 16):
                dsl = pl.ds(16 * k, 16)
                sidxo[dsl] = idxb[0, dsl] + src_off_v
                didxo[dsl] = idxb[1, dsl] + dst_off_v
                didxs[dsl] = idxb[1, dsl]

        def _issue_gathers(s):
            _, sidxo, didxo, _, elb, erb, hb, _, sem_ee, sem_h = slots[s][:10]
            pltpu.async_copy(el_h.at[sidxo], elb, sem_ee)
            pltpu.async_copy(er_h.at[didxo], erb, sem_ee)
            pltpu.async_copy(h_h.at[sidxo], hb, sem_h)

        def _wait_gathers_ee(s):
            _, sidxo, didxo, _, elb, erb, _, _, sem_ee = slots[s][:9]
            pltpu.make_async_copy(el_h.at[sidxo], elb, sem_ee).wait()
            pltpu.make_async_copy(er_h.at[didxo], erb, sem_ee).wait()

        def _wait_gather_h(s):
            _, sidxo, _, _, _, _, hb, _, _, sem_h = slots[s][:10]
            pltpu.make_async_copy(h_h.at[sidxo], hb, sem_h).wait()

        def _drain_scatters(s):
            _, _, _, didxs, _, _, hb, exb, _, _, sem_s = slots[s][:11]
            pltpu.make_async_copy(exb, den_sh.at[didxs], sem_s).wait()
            pltpu.make_async_copy(hb, t_sh.at[didxs], sem_s).wait()

        def _compute(s):
            _, _, _, didxs, elb, erb, hb, exb, _, _, sem_s = slots[s][:11]
            _wait_gathers_ee(s)

            @functools.partial(plsc.parallel_loop, 0, C // 2, unroll=2)
            def _ex(i):
                rows = 2 * i + row01
                el2 = plsc.load_gather(elb, [rows, col8])
                er2 = plsc.load_gather(erb, [rows, col8])
                z = el2 + er2 + eev
                ex = jnp.exp(jnp.where(z >= 0, z, NEG * z) - mv)
                plsc.store_scatter(exb, [rows, col8], ex)
            pltpu.async_copy(exb, den_sh.at[didxs], sem_s, add=True)
            _wait_gather_h(s)

            @functools.partial(plsc.parallel_loop, 0, C // 2)
            def _scale(p):
                rows = 2 * p + row01
                ex2 = plsc.load_gather(exb, [rows, col8])
                for q in range(2):
                    r = 2 * p + q
                    for hh in range(H):
                        sv = jnp.take(
                            ex2, jnp.full((16,), q * 8 + hh, jnp.int32),
                            mode="promise_in_bounds")
                        hb[r, pl.ds(16 * hh, 16)] = (
                            hb[r, pl.ds(16 * hh, 16)] * sv)
            pltpu.async_copy(hb, t_sh.at[didxs], sem_s, add=True)

        _issue_idx(0, 0).wait()
        _prep(0)
        _issue_gathers(0)
        _issue_idx(1, 1)

        npair = (NCHUNK + 1) // 2

        def _pair(jj, c):
            a = 2 * jj
            b = a + 1

            @pl.when(jj >= 1)
            def _():
                _drain_scatters(1)

            @pl.when(b < NCHUNK)
            def _():
                _wait_idx(1)
                _prep(1)
                _issue_gathers(1)
            _compute(0)

            @pl.when(jj < npair - 1)
            def _():
                _issue_idx(a + 2, 0)

            @pl.when(b < NCHUNK)
            def _():
                _compute(1)

                @pl.when(b + 2 < NCHUNK)
                def _():
                    _issue_idx(b + 2, 1)

            @pl.when(jj < npair - 1)
            def _():
                _drain_scatters(0)
                _wait_idx(0)
                _prep(0)
                _issue_gathers(0)
            return c
        lax.fori_loop(0, npair, _pair, 0)
        _drain_scatters(0)

    @pl.when(cid == 0)
    def _():
        _relation(e0_h, 0, N_NODE)

    @pl.when(cid == 1)
    def _():
        _relation(e1_h, N_NODE, 0)

    plsc.subcore_barrier()

    out_off = (1 - cid) * N_NODE

    def _wb(j, c):
        bb = j * NS + sid

        @pl.when(bb < NBLK)
        def _():
            rbase = bb * WBLK
            pltpu.sync_copy(t_sh.at[pl.ds(rbase, WBLK)], hb0)
            pltpu.sync_copy(den_sh.at[pl.ds(rbase, WBLK)], zb8)

            @functools.partial(plsc.parallel_loop, 0, WBLK // 2)
            def _div(p):
                rows = 2 * p + row01
                dv = jnp.maximum(plsc.load_gather(zb8, [rows, col8]), 1e-12)
                rv = 1.0 / dv
                for q in range(2):
                    r = 2 * p + q
                    for hh in range(H):
                        sv = jnp.take(
                            rv, jnp.full((16,), q * 8 + hh, jnp.int32),
                            mode="promise_in_bounds")
                        hb0[r, pl.ds(16 * hh, 16)] = (
                            hb0[r, pl.ds(16 * hh, 16)] * sv)
            pltpu.sync_copy(hb0, out_h.at[pl.ds(out_off + rbase, WBLK)])
        return c
    lax.fori_loop(0, JMAX, _wb, 0)


def _sc_messages(el_all, er_all, h_all, e0, e1, m8, ee8):
    mesh = plsc.VectorSubcoreMesh(core_axis_name="c", subcore_axis_name="s")
    slot = [
        pltpu.VMEM((2, C), jnp.int32),
        pltpu.VMEM((C,), jnp.int32),
        pltpu.VMEM((C,), jnp.int32),
        pltpu.VMEM((C,), jnp.int32),
        pltpu.VMEM((C, H), jnp.float32),
        pltpu.VMEM((C, H), jnp.float32),
        pltpu.VMEM((C, D), jnp.float32),
        pltpu.VMEM((C, H), jnp.float32),
    ]
    return pl.kernel(
        _sc_body,
        out_type=jax.ShapeDtypeStruct((N_ALL, D), jnp.float32),
        mesh=mesh,
        compiler_params=pltpu.CompilerParams(
            needs_layout_passes=False, use_tc_tiling_on_sc=False),
        scratch_types=[
            *slot, *slot,
            pltpu.VMEM((2, H), jnp.float32),
            pltpu.VMEM((2, H), jnp.float32),
            pltpu.VMEM((WBLK, H), jnp.float32),
            pltpu.VMEM_SHARED((N_NODE, D), jnp.float32),
            pltpu.VMEM_SHARED((N_NODE, H), jnp.float32),
            pltpu.SemaphoreType.DMA, pltpu.SemaphoreType.DMA,
            pltpu.SemaphoreType.DMA, pltpu.SemaphoreType.DMA,
            pltpu.SemaphoreType.DMA, pltpu.SemaphoreType.DMA,
            pltpu.SemaphoreType.DMA, pltpu.SemaphoreType.DMA,
        ],
    )(el_all, er_all, h_all, e0, e1, m8, ee8)


def _tc_post_body(tn_ref, h_ref, g_ref, b_ref, o_ref):
    y = _leaky_v(tn_ref[...] + h_ref[...])
    mu = jnp.mean(y, axis=1, keepdims=True)
    d = y - mu
    var = jnp.mean(d * d, axis=1, keepdims=True)
    o_ref[...] = g_ref[...] * d * lax.rsqrt(var + 1e-5) + b_ref[...]


def _tc_post(tn_all, h_all, gamma, beta):
    return pl.pallas_call(
        _tc_post_body,
        grid=(GRID,),
        in_specs=[
            pl.BlockSpec((ROWBLK, D), lambda i: (i, 0)),
            pl.BlockSpec((ROWBLK, D), lambda i: (i, 0)),
            pl.BlockSpec((1, D), lambda i: (0, 0)),
            pl.BlockSpec((1, D), lambda i: (0, 0)),
        ],
        out_specs=pl.BlockSpec((ROWBLK, D), lambda i: (i, 0)),
        out_shape=jax.ShapeDtypeStruct((N_ALL, D), jnp.float32),
    )(tn_all, h_all, gamma, beta)


def kernel(x_author, x_paper, edge_writes, edge_written_by, W_fc, W_fc_edge,
           edge_emb, attn_l, attn_r, attn_e, gamma, beta):
    x_all = jnp.concatenate([x_author, x_paper], axis=0)
    eye = jnp.eye(H, dtype=jnp.float32)
    al = (eye[:, None, :] * attn_l[0][:, :, None]).reshape(D, H)
    ar = (eye[:, None, :] * attn_r[0][:, :, None]).reshape(D, H)
    ae = (eye[:, None, :] * attn_e[0][:, :, None]).reshape(D, H)

    h_all, el_all, er_all, m8, ee8 = _tc_pre(
        x_all, W_fc, al, ar, ae, edge_emb, W_fc_edge)

    tn_all = _sc_messages(
        el_all, er_all, h_all, edge_writes, edge_written_by, m8, ee8)

    out_all = _tc_post(tn_all, h_all,
                       gamma.reshape(1, D), beta.reshape(1, D))
    return out_all[:N_NODE], out_all[N_NODE:]
